# baseline (speedup 1.0000x reference)
import jax
import jax.numpy as jnp
from jax import lax
from jax.experimental import pallas as pl
from jax.experimental.pallas import tpu as pltpu
from jax.experimental.pallas import tpu_sc as plsc

N = 1025
H = 16
RPC = 8
NW = 32
CHUNKS_PER_W = 4
UNROLL = 4


def _sc_body(idx_hbm, w_hbm, out_hbm, idx_v, out_v, wlut_v):
    cid = lax.axis_index("c")
    sid = lax.axis_index("s")
    wid = sid * 2 + cid

    pltpu.sync_copy(w_hbm, wlut_v)

    lanes = lax.iota(jnp.int32, 16)
    tail_cols = lanes + (N - 16)

    def do_rows(r0, nrows):
        pltpu.sync_copy(
            idx_hbm.at[pl.ds(r0, nrows), :],
            idx_v.at[pl.ds(0, nrows), :],
        )

        def head_body(h, carry):
            for r in range(nrows):
                def vec_body(v, c):
                    base = pl.multiple_of(v * 16 * UNROLL, 16 * UNROLL)
                    for k in range(UNROLL):
                        off = base + k * 16
                        out_v[r, pl.ds(off, 16)] = plsc.load_gather(
                            wlut_v, [idx_v[r, pl.ds(off, 16)] * H + h]
                        )
                    return c

                lax.fori_loop(0, (N // 16) // UNROLL, vec_body, 0)
                rows16 = jnp.full((16,), r, jnp.int32)
                vec = plsc.load_gather(idx_v, [rows16, tail_cols])
                vals = plsc.load_gather(wlut_v, [vec * H + h])
                plsc.store_scatter(out_v, [rows16, tail_cols], vals)
            pltpu.sync_copy(
                out_v.at[pl.ds(0, nrows), :],
                out_hbm.at[h, pl.ds(r0, nrows), :],
            )
            return carry

        lax.fori_loop(0, H, head_body, 0)

    def chunk_body(c, carry):
        do_rows((wid * CHUNKS_PER_W + c) * RPC, RPC)
        return carry

    lax.fori_loop(0, CHUNKS_PER_W, chunk_body, 0)

    @pl.when(wid == NW - 1)
    def _():
        do_rows(N - 1, 1)


def kernel(spatial_bias, weight):
    wflat = weight.reshape(-1)
    mesh = plsc.VectorSubcoreMesh(core_axis_name="c", subcore_axis_name="s")
    run = pl.kernel(
        _sc_body,
        mesh=mesh,
        compiler_params=pltpu.CompilerParams(needs_layout_passes=False),
        out_type=jax.ShapeDtypeStruct((H, N, N), jnp.float32),
        scratch_types=[
            pltpu.VMEM((RPC, N), jnp.int32),
            pltpu.VMEM((RPC, N), jnp.float32),
            pltpu.VMEM((2 * 32 * H,), jnp.float32),
        ],
    )
    out = run(spatial_bias, wflat)
    return out.reshape(1, H, N, N)


# speedup vs baseline: 1.7213x; 1.7213x over previous
import jax
import jax.numpy as jnp
from jax import lax
from jax.experimental import pallas as pl
from jax.experimental.pallas import tpu as pltpu
from jax.experimental.pallas import tpu_sc as plsc

N = 1025
H = 16
RPC = 8
NW = 32
CHUNKS_PER_W = 4
UNROLL = 4


def _sc_body(idx_hbm, w_hbm, out_hbm, idx_v, out_v, wlut_v):
    cid = lax.axis_index("c")
    sid = lax.axis_index("s")
    wid = sid * 2 + cid

    pltpu.sync_copy(w_hbm, wlut_v)

    lanes = lax.iota(jnp.int32, 16)
    tail_cols = lanes + (N - 16)

    def do_rows(r0, nrows):
        pltpu.sync_copy(
            idx_hbm.at[pl.ds(r0, nrows), :],
            idx_v.at[pl.ds(0, nrows), :],
        )

        def head_body(h, carry):
            for r in range(nrows):
                @plsc.parallel_loop(0, N // 16, unroll=UNROLL)
                def vec_body(v):
                    off = pl.multiple_of(v * 16, 16)
                    out_v[r, pl.ds(off, 16)] = plsc.load_gather(
                        wlut_v, [idx_v[r, pl.ds(off, 16)] * H + h]
                    )
                rows16 = jnp.full((16,), r, jnp.int32)
                vec = plsc.load_gather(idx_v, [rows16, tail_cols])
                vals = plsc.load_gather(wlut_v, [vec * H + h])
                plsc.store_scatter(out_v, [rows16, tail_cols], vals)
            pltpu.sync_copy(
                out_v.at[pl.ds(0, nrows), :],
                out_hbm.at[h, pl.ds(r0, nrows), :],
            )
            return carry

        lax.fori_loop(0, H, head_body, 0)

    def chunk_body(c, carry):
        do_rows((wid * CHUNKS_PER_W + c) * RPC, RPC)
        return carry

    lax.fori_loop(0, CHUNKS_PER_W, chunk_body, 0)

    @pl.when(wid == NW - 1)
    def _():
        do_rows(N - 1, 1)


def kernel(spatial_bias, weight):
    wflat = weight.reshape(-1)
    mesh = plsc.VectorSubcoreMesh(core_axis_name="c", subcore_axis_name="s")
    run = pl.kernel(
        _sc_body,
        mesh=mesh,
        compiler_params=pltpu.CompilerParams(needs_layout_passes=False),
        out_type=jax.ShapeDtypeStruct((H, N, N), jnp.float32),
        scratch_types=[
            pltpu.VMEM((RPC, N), jnp.int32),
            pltpu.VMEM((RPC, N), jnp.float32),
            pltpu.VMEM((2 * 32 * H,), jnp.float32),
        ],
    )
    out = run(spatial_bias, wflat)
    return out.reshape(1, H, N, N)


# speedup vs baseline: 1.7573x; 1.0209x over previous
import jax
import jax.numpy as jnp
from jax import lax
from jax.experimental import pallas as pl
from jax.experimental.pallas import tpu as pltpu
from jax.experimental.pallas import tpu_sc as plsc

N = 1025
H = 16
RPC = 8
NW = 32
CHUNKS_PER_W = 4
UNROLL = 4


def _sc_body(idx_hbm, w_hbm, out_hbm, idx_v, out_v, wlut_v):
    cid = lax.axis_index("c")
    sid = lax.axis_index("s")
    wid = sid * 2 + cid

    pltpu.sync_copy(w_hbm, wlut_v)

    lanes = lax.iota(jnp.int32, 16)
    tail_cols = lanes + (N - 16)

    def do_rows(r0, nrows):
        pltpu.sync_copy(
            idx_hbm.at[pl.ds(r0, nrows), :],
            idx_v.at[pl.ds(0, nrows), :],
        )

        def head_body(h, carry):
            @plsc.parallel_loop(0, nrows * (N // 16), unroll=UNROLL)
            def vec_body(i):
                r = i // (N // 16)
                off = pl.multiple_of((i % (N // 16)) * 16, 16)
                out_v[r, pl.ds(off, 16)] = plsc.load_gather(
                    wlut_v, [idx_v[r, pl.ds(off, 16)] * H + h]
                )
            for r in range(nrows):
                rows16 = jnp.full((16,), r, jnp.int32)
                vec = plsc.load_gather(idx_v, [rows16, tail_cols])
                vals = plsc.load_gather(wlut_v, [vec * H + h])
                plsc.store_scatter(out_v, [rows16, tail_cols], vals)
            pltpu.sync_copy(
                out_v.at[pl.ds(0, nrows), :],
                out_hbm.at[h, pl.ds(r0, nrows), :],
            )
            return carry

        lax.fori_loop(0, H, head_body, 0)

    def chunk_body(c, carry):
        do_rows((wid * CHUNKS_PER_W + c) * RPC, RPC)
        return carry

    lax.fori_loop(0, CHUNKS_PER_W, chunk_body, 0)

    @pl.when(wid == NW - 1)
    def _():
        do_rows(N - 1, 1)


def kernel(spatial_bias, weight):
    wflat = weight.reshape(-1)
    mesh = plsc.VectorSubcoreMesh(core_axis_name="c", subcore_axis_name="s")
    run = pl.kernel(
        _sc_body,
        mesh=mesh,
        compiler_params=pltpu.CompilerParams(needs_layout_passes=False),
        out_type=jax.ShapeDtypeStruct((H, N, N), jnp.float32),
        scratch_types=[
            pltpu.VMEM((RPC, N), jnp.int32),
            pltpu.VMEM((RPC, N), jnp.float32),
            pltpu.VMEM((2 * 32 * H,), jnp.float32),
        ],
    )
    out = run(spatial_bias, wflat)
    return out.reshape(1, H, N, N)
